# baseline (speedup 1.0000x reference)
import jax
import jax.numpy as jnp
from jax import lax
from jax.experimental import pallas as pl
from jax.experimental.pallas import tpu as pltpu, tpu_sc as plsc

BATCH = 16384
EMBED_K = 128
CHUNK = 128
NBUF = 7

_info = plsc.get_sparse_core_info()
NC, NS, NL = _info.num_cores, _info.num_subcores, _info.num_lanes
NW = NC * NS
B_PER_W = BATCH // NW
CHUNKS_PER_W = B_PER_W // CHUNK

_mesh = plsc.VectorSubcoreMesh(core_axis_name="c", subcore_axis_name="s")


@jax.jit
def _gather2(x_flat, user_table, item_table):
    @pl.kernel(
        mesh=_mesh,
        compiler_params=pltpu.CompilerParams(needs_layout_passes=False),
        out_type=(
            jax.ShapeDtypeStruct((BATCH, EMBED_K), jnp.float32),
            jax.ShapeDtypeStruct((BATCH, EMBED_K), jnp.float32),
        ),
        scratch_types=[
            pltpu.VMEM((2 * B_PER_W // CHUNK, CHUNK), jnp.int32),
            pltpu.VMEM((CHUNKS_PER_W, CHUNK), jnp.int32),
            pltpu.VMEM((CHUNKS_PER_W, CHUNK), jnp.int32),
            pltpu.VMEM((NBUF, CHUNK, EMBED_K), jnp.float32),
            pltpu.SemaphoreType.DMA((NBUF,)),
            pltpu.SemaphoreType.DMA((NBUF,)),
            pltpu.SemaphoreType.DMA,
        ],
    )
    def k(x_hbm, utab_hbm, itab_hbm, uout_hbm, iout_hbm,
          xbuf, idx_u, idx_i, rows, gsem, osem, isem):
        wid = lax.axis_index("s") * NC + lax.axis_index("c")
        base = wid * B_PER_W
        xrows = 2 * B_PER_W // CHUNK
        pltpu.async_copy(
            x_hbm.at[pl.ds(wid * xrows, xrows)], xbuf, isem).wait()

        lane = lax.iota(jnp.int32, NL)
        for t in range(B_PER_W // NL):
            srow = jnp.full((NL,), (2 * t * NL) // CHUNK, jnp.int32)
            scol = (2 * t * NL) % CHUNK + 2 * lane
            r, o = divmod(t * NL, CHUNK)
            idx_u[r, pl.ds(o, NL)] = plsc.load_gather(xbuf, [srow, scol])
            idx_i[r, pl.ds(o, NL)] = plsc.load_gather(xbuf, [srow, scol + 1])

        def fire_gather(c, buf):
            if c < CHUNKS_PER_W:
                src = utab_hbm.at[idx_u.at[c]]
            else:
                src = itab_hbm.at[idx_i.at[c - CHUNKS_PER_W]]
            return pltpu.async_copy(src, rows.at[buf], gsem.at[buf])

        def fire_out(c, buf):
            if c < CHUNKS_PER_W:
                dst = uout_hbm.at[pl.ds(base + c * CHUNK, CHUNK)]
            else:
                dst = iout_hbm.at[pl.ds(base + (c - CHUNKS_PER_W) * CHUNK, CHUNK)]
            return pltpu.async_copy(rows.at[buf], dst, osem.at[buf])

        nchunks = 2 * CHUNKS_PER_W
        gathers = [fire_gather(c, c % NBUF) for c in range(NBUF)]
        outs = [None] * NBUF
        for c in range(nchunks):
            buf = c % NBUF
            gathers[buf].wait()
            outs[buf] = fire_out(c, buf)
            if c + NBUF < nchunks:
                outs[buf].wait()
                gathers[buf] = fire_gather(c + NBUF, buf)
        for c in range(nchunks - NBUF, nchunks):
            outs[c % NBUF].wait()

    return k(x_flat, user_table, item_table)


def kernel(x, user_table, item_table):
    return _gather2(x.reshape(-1, CHUNK), user_table, item_table)


# speedup vs baseline: 1.2885x; 1.2885x over previous
import jax
import jax.numpy as jnp
from jax import lax
from jax.experimental import pallas as pl
from jax.experimental.pallas import tpu as pltpu, tpu_sc as plsc

BATCH = 16384
EMBED_K = 128
CHUNK = 256
NBUF = 3
IDXROW = 128

_info = plsc.get_sparse_core_info()
NC, NS = _info.num_cores, _info.num_subcores
NW = NC * NS
B_PER_W = BATCH // NW
CHUNKS_PER_W = B_PER_W // CHUNK

_mesh = plsc.VectorSubcoreMesh(core_axis_name="c", subcore_axis_name="s")


@jax.jit
def _gather2(user_idx, item_idx, user_table, item_table):
    @pl.kernel(
        mesh=_mesh,
        out_type=(
            jax.ShapeDtypeStruct((BATCH, EMBED_K), jnp.float32),
            jax.ShapeDtypeStruct((BATCH, EMBED_K), jnp.float32),
        ),
        scratch_types=[
            pltpu.VMEM((B_PER_W,), jnp.int32),
            pltpu.VMEM((B_PER_W,), jnp.int32),
            pltpu.VMEM((NBUF, CHUNK, EMBED_K), jnp.float32),
            pltpu.SemaphoreType.DMA((NBUF,)),
            pltpu.SemaphoreType.DMA((NBUF,)),
            pltpu.SemaphoreType.DMA,
        ],
    )
    def k(uidx_hbm, iidx_hbm, utab_hbm, itab_hbm, uout_hbm, iout_hbm,
          idx_u, idx_i, rows, gsem, osem, isem):
        wid = lax.axis_index("s") * NC + lax.axis_index("c")
        base = wid * B_PER_W
        iu = pltpu.async_copy(uidx_hbm.at[pl.ds(base, B_PER_W)], idx_u, isem)
        ii = pltpu.async_copy(iidx_hbm.at[pl.ds(base, B_PER_W)], idx_i, isem)
        iu.wait()
        ii.wait()

        def fire_gather(c, buf):
            if c < CHUNKS_PER_W:
                src = utab_hbm.at[idx_u.at[pl.ds(c * CHUNK, CHUNK)]]
            else:
                src = itab_hbm.at[idx_i.at[pl.ds((c - CHUNKS_PER_W) * CHUNK, CHUNK)]]
            return pltpu.async_copy(src, rows.at[buf], gsem.at[buf])

        def fire_out(c, buf):
            if c < CHUNKS_PER_W:
                dst = uout_hbm.at[pl.ds(base + c * CHUNK, CHUNK)]
            else:
                dst = iout_hbm.at[pl.ds(base + (c - CHUNKS_PER_W) * CHUNK, CHUNK)]
            return pltpu.async_copy(rows.at[buf], dst, osem.at[buf])

        nchunks = 2 * CHUNKS_PER_W
        gathers = [fire_gather(c, c % NBUF) for c in range(min(NBUF, nchunks))]
        outs = [None] * NBUF
        for c in range(nchunks):
            buf = c % NBUF
            gathers[buf].wait()
            outs[buf] = fire_out(c, buf)
            if c + NBUF < nchunks:
                outs[buf].wait()
                gathers[buf] = fire_gather(c + NBUF, buf)
        for c in range(max(0, nchunks - NBUF), nchunks):
            outs[c % NBUF].wait()

    return k(user_idx, item_idx, user_table, item_table)


def kernel(x, user_table, item_table):
    return _gather2(x[:, 0], x[:, 1], user_table, item_table)


# speedup vs baseline: 1.2992x; 1.0083x over previous
import jax
import jax.numpy as jnp
from jax import lax
from jax.experimental import pallas as pl
from jax.experimental.pallas import tpu as pltpu, tpu_sc as plsc

BATCH = 16384
EMBED_K = 128
CHUNK = 128
NBUF = 7

_info = plsc.get_sparse_core_info()
NC, NS = _info.num_cores, _info.num_subcores
NW = NC * NS
B_PER_W = BATCH // NW
CHUNKS_PER_W = B_PER_W // CHUNK

_mesh = plsc.VectorSubcoreMesh(core_axis_name="c", subcore_axis_name="s")


@jax.jit
def _gather2(user_idx, item_idx, user_table, item_table):
    @pl.kernel(
        mesh=_mesh,
        out_type=(
            jax.ShapeDtypeStruct((BATCH, EMBED_K), jnp.float32),
            jax.ShapeDtypeStruct((BATCH, EMBED_K), jnp.float32),
        ),
        scratch_types=[
            pltpu.VMEM((B_PER_W,), jnp.int32),
            pltpu.VMEM((B_PER_W,), jnp.int32),
            pltpu.VMEM((NBUF, CHUNK, EMBED_K), jnp.float32),
            pltpu.SemaphoreType.DMA((NBUF,)),
            pltpu.SemaphoreType.DMA((NBUF,)),
            pltpu.SemaphoreType.DMA,
        ],
    )
    def k(uidx_hbm, iidx_hbm, utab_hbm, itab_hbm, uout_hbm, iout_hbm,
          idx_u, idx_i, rows, gsem, osem, isem):
        wid = lax.axis_index("s") * NC + lax.axis_index("c")
        base = wid * B_PER_W
        iu = pltpu.async_copy(uidx_hbm.at[pl.ds(base, B_PER_W)], idx_u, isem)
        ii = pltpu.async_copy(iidx_hbm.at[pl.ds(base, B_PER_W)], idx_i, isem)
        iu.wait()
        ii.wait()

        def fire_gather(c, buf):
            tab, j = (utab_hbm, idx_u) if c % 2 == 0 else (itab_hbm, idx_i)
            src = tab.at[j.at[pl.ds((c // 2) * CHUNK, CHUNK)]]
            return pltpu.async_copy(src, rows.at[buf], gsem.at[buf])

        def fire_out(c, buf):
            out = uout_hbm if c % 2 == 0 else iout_hbm
            dst = out.at[pl.ds(base + (c // 2) * CHUNK, CHUNK)]
            return pltpu.async_copy(rows.at[buf], dst, osem.at[buf])

        nchunks = 2 * CHUNKS_PER_W
        gathers = [fire_gather(c, c % NBUF) for c in range(min(NBUF, nchunks))]
        outs = [None] * NBUF
        for c in range(nchunks):
            buf = c % NBUF
            gathers[buf].wait()
            outs[buf] = fire_out(c, buf)
            if c + NBUF < nchunks:
                outs[buf].wait()
                gathers[buf] = fire_gather(c + NBUF, buf)
        for c in range(max(0, nchunks - NBUF), nchunks):
            outs[c % NBUF].wait()

    return k(user_idx, item_idx, user_table, item_table)


def kernel(x, user_table, item_table):
    return _gather2(x[:, 0], x[:, 1], user_table, item_table)
